# baseline (speedup 1.0000x reference)
import jax
import jax.numpy as jnp
from jax import lax
from jax.experimental import pallas as pl
from jax.experimental.pallas import tpu as pltpu
from jax.experimental.pallas import tpu_sc as plsc

DIM = 32
FIELDS = 26
NCAT = 1000000
NTILE_FULL = 7812
NPAIRS1 = 122
NF = 13
BBLK = 128
NBLK = 8
NPAIR2 = 6


def _transpose_kernel(tt_hbm, tail_hbm, t4_hbm,
                      in0, in1, out0, out1, isem0, isem1, osem0, osem1):
    w = lax.axis_index("s") * 2 + lax.axis_index("c")
    iota = lax.iota(jnp.int32, 16)
    jt16 = (iota, iota + 16)
    ins = (in0, in1)
    outs = (out0, out1)
    isem = (isem0, isem1)
    osem = (osem0, osem1)

    def fire_in(n, pb):
        it = w + 32 * n
        i0 = pl.multiple_of(it * 128, 128)
        for jt in range(4):
            pltpu.async_copy(tt_hbm.at[pl.ds(jt * 8, 8), pl.ds(i0, 128)],
                             ins[pb].at[jt], isem[pb])

    def drain_in(n, pb):
        it = w + 32 * n
        i0 = pl.multiple_of(it * 128, 128)
        for jt in range(4):
            pltpu.make_async_copy(
                tt_hbm.at[pl.ds(jt * 8, 8), pl.ds(i0, 128)],
                ins[pb].at[jt], isem[pb]).wait()

    def extract(pb, ncat):
        in_f = ins[pb]
        out_f = outs[pb]
        for half in range(2):
            dv = jt16[half]
            a16 = lax.shift_right_logical(dv, 3)
            b16 = jnp.bitwise_and(dv, 7)
            for c in range(ncat):
                c16 = jnp.full((16,), c, jnp.int32)
                vals = plsc.load_gather(in_f, [a16, b16, c16])
                r = (c * 32 + half * 16) // 128
                col = (c * 32 + half * 16) % 128
                out_f[r, pl.ds(col, 16)] = vals

    def fire_out(n, pb):
        it = w + 32 * n
        pltpu.async_copy(outs[pb], t4_hbm.at[pl.ds(it * 32, 32), :], osem[pb])

    def wait_out(n, pb):
        it = w + 32 * n
        pltpu.make_async_copy(outs[pb], t4_hbm.at[pl.ds(it * 32, 32), :],
                              osem[pb]).wait()

    fire_in(0, 0)

    def pair_body(i, carry):
        n0 = 2 * i
        fire_in(n0 + 1, 1)
        drain_in(n0, 0)

        @pl.when(i > 0)
        def _():
            wait_out(n0 - 2, 0)
        extract(0, 128)
        fire_out(n0, 0)

        @pl.when(i < NPAIRS1 - 1)
        def _():
            fire_in(n0 + 2, 0)
        drain_in(n0 + 1, 1)

        @pl.when(i > 0)
        def _():
            wait_out(n0 - 1, 1)
        extract(1, 128)
        fire_out(n0 + 1, 1)
        return carry

    lax.fori_loop(0, NPAIRS1, pair_body, 0)
    wait_out(2 * NPAIRS1 - 2, 0)
    wait_out(2 * NPAIRS1 - 1, 1)

    @pl.when(w < 4)
    def _():
        fire_in(244, 0)
        drain_in(244, 0)
        extract(0, 128)
        fire_out(244, 0)
        wait_out(244, 0)

    @pl.when(w == 4)
    def _():
        pltpu.sync_copy(tail_hbm, out0.at[pl.ds(0, 16), :])
        pltpu.sync_copy(out0.at[pl.ds(0, 16), :],
                        t4_hbm.at[pl.ds(NTILE_FULL * 32, 16), :])


def _gather_kernel(tl_hbm, idx_hbm, out_hbm,
                   xi_v, r0, r1, rows0, rows1, stage_v, gsem0, gsem1):
    w = lax.axis_index("s") * 2 + lax.axis_index("c")
    h = w // 16
    bg = w % 16
    f0 = h * NF
    rbuf = (r0, r1)
    rows = (rows0, rows1)
    gsem = (gsem0, gsem1)
    iota = lax.iota(jnp.int32, 16)

    def build_r(f, pb):
        r_v = rbuf[pb]

        def g_body(g, carry):
            p16 = (iota + g * 16) * FIELDS + (f0 + f)
            r_v[pl.ds(g * 16, 16)] = plsc.load_gather(xi_v, [p16])
            return carry
        lax.fori_loop(0, BBLK // 16, g_body, 0)

    def issue(pb):
        pltpu.async_copy(tl_hbm.at[rbuf[pb]], rows[pb], gsem[pb])

    def wait(pb):
        pltpu.make_async_copy(tl_hbm.at[rbuf[pb]], rows[pb], gsem[pb]).wait()

    def extract(f, bt, pb):
        rows_b = rows[pb]
        for g in range(BBLK // 16):
            b16 = iota + g * 16
            for d in range(DIM):
                stage_v[d // 8, d % 8, pl.ds(g * 16, 16)] = plsc.load_gather(
                    rows_b, [b16, jnp.full((16,), d, jnp.int32)])
        pltpu.sync_copy(stage_v, out_hbm.at[f0 + f, :, bt, :, :])

    def block_body(blk, carry):
        bb0 = bg * (NBLK * BBLK) + blk * BBLK
        bt = bg * NBLK + blk
        pltpu.sync_copy(idx_hbm.at[pl.ds(bb0 * FIELDS, BBLK * FIELDS)], xi_v)

        build_r(0, 0)
        issue(0)

        def pair_body(i, carry2):
            c = 2 * i
            build_r(c + 1, 1)
            issue(1)
            wait(0)
            extract(c, bt, 0)
            build_r(c + 2, 0)
            issue(0)
            wait(1)
            extract(c + 1, bt, 1)
            return carry2
        lax.fori_loop(0, NPAIR2, pair_body, 0)

        wait(0)
        extract(NF - 1, bt, 0)
        return carry

    lax.fori_loop(0, NBLK, block_body, 0)


def kernel(x, embedding):
    batch, fields = x.shape
    b = batch * fields
    tt = jnp.transpose(embedding)
    mesh = plsc.VectorSubcoreMesh(core_axis_name="c", subcore_axis_name="s")

    k1 = pl.kernel(
        _transpose_kernel,
        out_type=jax.ShapeDtypeStruct((250000, 128), jnp.float32),
        mesh=mesh,
        scratch_types=[
            pltpu.VMEM((4, 8, 128), jnp.float32),
            pltpu.VMEM((4, 8, 128), jnp.float32),
            pltpu.VMEM((32, 128), jnp.float32),
            pltpu.VMEM((32, 128), jnp.float32),
            pltpu.SemaphoreType.DMA,
            pltpu.SemaphoreType.DMA,
            pltpu.SemaphoreType.DMA,
            pltpu.SemaphoreType.DMA,
        ],
        compiler_params=pltpu.CompilerParams(
            use_tc_tiling_on_sc=True, needs_layout_passes=False),
    )
    tail16 = embedding[NTILE_FULL * 128:, :].reshape(16, 128)
    t4 = k1(tt, tail16)
    tl = t4.reshape(NCAT, DIM)

    k2 = pl.kernel(
        _gather_kernel,
        out_type=jax.ShapeDtypeStruct((FIELDS, 4, 128, 8, 128), jnp.float32),
        mesh=mesh,
        scratch_types=[
            pltpu.VMEM((BBLK * FIELDS,), jnp.int32),
            pltpu.VMEM((BBLK,), jnp.int32),
            pltpu.VMEM((BBLK,), jnp.int32),
            pltpu.VMEM((BBLK, DIM), jnp.float32),
            pltpu.VMEM((BBLK, DIM), jnp.float32),
            pltpu.VMEM((4, 8, BBLK), jnp.float32),
            pltpu.SemaphoreType.DMA,
            pltpu.SemaphoreType.DMA,
        ],
        compiler_params=pltpu.CompilerParams(
            use_tc_tiling_on_sc=False, needs_layout_passes=False),
    )
    out5 = k2(tl, x.reshape(b))
    return jnp.transpose(out5, (2, 4, 0, 1, 3)).reshape(batch, fields, DIM)


# speedup vs baseline: 2.0072x; 2.0072x over previous
import jax
import jax.numpy as jnp
from jax import lax
from jax.experimental import pallas as pl
from jax.experimental.pallas import tpu as pltpu
from jax.experimental.pallas import tpu_sc as plsc

DIM = 32
FIELDS = 26
NCAT = 1000000
NTILE_FULL = 7812
NPAIRS1 = 122
NF = 13
BBLK = 128
NBLK = 8
NPAIR2 = 6


def _transpose_kernel(tt_hbm, tail_hbm, t4_hbm,
                      in0, in1, out0, out1, isem0, isem1, osem0, osem1):
    w = lax.axis_index("s") * 2 + lax.axis_index("c")
    iota = lax.iota(jnp.int32, 16)
    jt16 = (iota, iota + 16)
    ins = (in0, in1)
    outs = (out0, out1)
    isem = (isem0, isem1)
    osem = (osem0, osem1)

    def fire_in(n, pb):
        it = w + 32 * n
        i0 = pl.multiple_of(it * 128, 128)
        for jt in range(4):
            pltpu.async_copy(tt_hbm.at[pl.ds(jt * 8, 8), pl.ds(i0, 128)],
                             ins[pb].at[pl.ds(jt * 8, 8), :], isem[pb])

    def drain_in(n, pb):
        it = w + 32 * n
        i0 = pl.multiple_of(it * 128, 128)
        for jt in range(4):
            pltpu.make_async_copy(
                tt_hbm.at[pl.ds(jt * 8, 8), pl.ds(i0, 128)],
                ins[pb].at[pl.ds(jt * 8, 8), :], isem[pb]).wait()

    def extract(pb):
        in_f = ins[pb]
        out_f = outs[pb]

        @plsc.parallel_loop(0, 128, unroll=8)
        def _(c):
            c16 = jnp.full((16,), 0, jnp.int32) + c
            for half in range(2):
                vals = plsc.load_gather(in_f, [jt16[half], c16])
                pos = c * 32 + half * 16
                r = lax.shift_right_logical(pos, 7)
                col = pl.multiple_of(jnp.bitwise_and(pos, 127), 16)
                out_f[r, pl.ds(col, 16)] = vals

    def fire_out(n, pb):
        it = w + 32 * n
        pltpu.async_copy(outs[pb], t4_hbm.at[pl.ds(it * 32, 32), :], osem[pb])

    def wait_out(n, pb):
        it = w + 32 * n
        pltpu.make_async_copy(outs[pb], t4_hbm.at[pl.ds(it * 32, 32), :],
                              osem[pb]).wait()

    fire_in(0, 0)

    def pair_body(i, carry):
        n0 = 2 * i
        fire_in(n0 + 1, 1)
        drain_in(n0, 0)

        @pl.when(i > 0)
        def _():
            wait_out(n0 - 2, 0)
        extract(0)
        fire_out(n0, 0)

        @pl.when(i < NPAIRS1 - 1)
        def _():
            fire_in(n0 + 2, 0)
        drain_in(n0 + 1, 1)

        @pl.when(i > 0)
        def _():
            wait_out(n0 - 1, 1)
        extract(1)
        fire_out(n0 + 1, 1)
        return carry

    lax.fori_loop(0, NPAIRS1, pair_body, 0)
    wait_out(2 * NPAIRS1 - 2, 0)
    wait_out(2 * NPAIRS1 - 1, 1)

    @pl.when(w < 4)
    def _():
        fire_in(244, 0)
        drain_in(244, 0)
        extract(0)
        fire_out(244, 0)
        wait_out(244, 0)

    @pl.when(w == 4)
    def _():
        pltpu.sync_copy(tail_hbm, out0.at[pl.ds(0, 16), :])
        pltpu.sync_copy(out0.at[pl.ds(0, 16), :],
                        t4_hbm.at[pl.ds(NTILE_FULL * 32, 16), :])


def _gather_kernel(tl_hbm, idx_hbm, out_hbm,
                   xi_v, r0, r1, rows0, rows1, stage_v, gsem0, gsem1):
    w = lax.axis_index("s") * 2 + lax.axis_index("c")
    h = w // 16
    bg = w % 16
    f0 = h * NF
    rbuf = (r0, r1)
    rows = (rows0, rows1)
    gsem = (gsem0, gsem1)
    iota = lax.iota(jnp.int32, 16)

    def build_r(f, pb):
        r_v = rbuf[pb]

        @plsc.parallel_loop(0, BBLK // 16, unroll=4)
        def _(g):
            g16 = pl.multiple_of(g * 16, 16)
            p16 = (iota + g16) * FIELDS + (f0 + f)
            r_v[pl.ds(g16, 16)] = plsc.load_gather(xi_v, [p16])

    def issue(pb):
        pltpu.async_copy(tl_hbm.at[rbuf[pb]], rows[pb], gsem[pb])

    def wait(pb):
        pltpu.make_async_copy(tl_hbm.at[rbuf[pb]], rows[pb], gsem[pb]).wait()

    def extract(f, bt, pb):
        rows_b = rows[pb]

        @plsc.parallel_loop(0, DIM, unroll=4)
        def _(d):
            d16 = jnp.full((16,), 0, jnp.int32) + d
            dr = lax.shift_right_logical(d, 3)
            dj = jnp.bitwise_and(d, 7)
            for g in range(BBLK // 16):
                stage_v[dr, dj, pl.ds(g * 16, 16)] = plsc.load_gather(
                    rows_b, [iota + g * 16, d16])
        pltpu.sync_copy(stage_v, out_hbm.at[f0 + f, :, bt, :, :])

    def block_body(blk, carry):
        bb0 = bg * (NBLK * BBLK) + blk * BBLK
        bt = bg * NBLK + blk
        pltpu.sync_copy(idx_hbm.at[pl.ds(bb0 * FIELDS, BBLK * FIELDS)], xi_v)

        build_r(0, 0)
        issue(0)

        def pair_body(i, carry2):
            c = 2 * i
            build_r(c + 1, 1)
            issue(1)
            wait(0)
            extract(c, bt, 0)
            build_r(c + 2, 0)
            issue(0)
            wait(1)
            extract(c + 1, bt, 1)
            return carry2
        lax.fori_loop(0, NPAIR2, pair_body, 0)

        wait(0)
        extract(NF - 1, bt, 0)
        return carry

    lax.fori_loop(0, NBLK, block_body, 0)


def kernel(x, embedding):
    batch, fields = x.shape
    b = batch * fields
    tt = jnp.transpose(embedding)
    mesh = plsc.VectorSubcoreMesh(core_axis_name="c", subcore_axis_name="s")

    k1 = pl.kernel(
        _transpose_kernel,
        out_type=jax.ShapeDtypeStruct((250000, 128), jnp.float32),
        mesh=mesh,
        scratch_types=[
            pltpu.VMEM((32, 128), jnp.float32),
            pltpu.VMEM((32, 128), jnp.float32),
            pltpu.VMEM((32, 128), jnp.float32),
            pltpu.VMEM((32, 128), jnp.float32),
            pltpu.SemaphoreType.DMA,
            pltpu.SemaphoreType.DMA,
            pltpu.SemaphoreType.DMA,
            pltpu.SemaphoreType.DMA,
        ],
        compiler_params=pltpu.CompilerParams(
            use_tc_tiling_on_sc=True, needs_layout_passes=False),
    )
    tail16 = embedding[NTILE_FULL * 128:, :].reshape(16, 128)
    t4 = k1(tt, tail16)
    tl = t4.reshape(NCAT, DIM)

    k2 = pl.kernel(
        _gather_kernel,
        out_type=jax.ShapeDtypeStruct((FIELDS, 4, 128, 8, 128), jnp.float32),
        mesh=mesh,
        scratch_types=[
            pltpu.VMEM((BBLK * FIELDS,), jnp.int32),
            pltpu.VMEM((BBLK,), jnp.int32),
            pltpu.VMEM((BBLK,), jnp.int32),
            pltpu.VMEM((BBLK, DIM), jnp.float32),
            pltpu.VMEM((BBLK, DIM), jnp.float32),
            pltpu.VMEM((4, 8, BBLK), jnp.float32),
            pltpu.SemaphoreType.DMA,
            pltpu.SemaphoreType.DMA,
        ],
        compiler_params=pltpu.CompilerParams(
            use_tc_tiling_on_sc=False, needs_layout_passes=False),
    )
    out5 = k2(tl, x.reshape(b))
    return jnp.transpose(out5, (2, 4, 0, 1, 3)).reshape(batch, fields, DIM)


# speedup vs baseline: 2.0719x; 1.0322x over previous
import jax
import jax.numpy as jnp
from jax import lax
from jax.experimental import pallas as pl
from jax.experimental.pallas import tpu as pltpu
from jax.experimental.pallas import tpu_sc as plsc

DIM = 32
FIELDS = 26
NCAT = 1000000
NTILE_FULL = 7812
NPAIRS1 = 122
NF = 13
BBLK = 128
NBLK = 8
NPAIR2 = 6


def _transpose_kernel(tt_hbm, tail_hbm, t4_hbm,
                      in0, in1, out0, out1, isem0, isem1, osem0, osem1):
    w = lax.axis_index("s") * 2 + lax.axis_index("c")
    iota = lax.iota(jnp.int32, 16)
    jt16 = (iota, iota + 16)
    ins = (in0, in1)
    outs = (out0, out1)
    isem = (isem0, isem1)
    osem = (osem0, osem1)

    def fire_in(n, pb):
        it = w + 32 * n
        i0 = pl.multiple_of(it * 128, 128)
        pltpu.async_copy(tt_hbm.at[:, pl.ds(i0, 128)], ins[pb], isem[pb])

    def drain_in(n, pb):
        it = w + 32 * n
        i0 = pl.multiple_of(it * 128, 128)
        pltpu.make_async_copy(tt_hbm.at[:, pl.ds(i0, 128)], ins[pb],
                              isem[pb]).wait()

    def extract(pb):
        in_f = ins[pb]
        out_f = outs[pb]

        @plsc.parallel_loop(0, 128, unroll=8)
        def _(c):
            c16 = jnp.full((16,), 0, jnp.int32) + c
            for half in range(2):
                vals = plsc.load_gather(in_f, [jt16[half], c16])
                pos = c * 32 + half * 16
                r = lax.shift_right_logical(pos, 7)
                col = pl.multiple_of(jnp.bitwise_and(pos, 127), 16)
                out_f[r, pl.ds(col, 16)] = vals

    def fire_out(n, pb):
        it = w + 32 * n
        pltpu.async_copy(outs[pb], t4_hbm.at[pl.ds(it * 32, 32), :], osem[pb])

    def wait_out(n, pb):
        it = w + 32 * n
        pltpu.make_async_copy(outs[pb], t4_hbm.at[pl.ds(it * 32, 32), :],
                              osem[pb]).wait()

    fire_in(0, 0)

    def pair_body(i, carry):
        n0 = 2 * i
        fire_in(n0 + 1, 1)
        drain_in(n0, 0)

        @pl.when(i > 0)
        def _():
            wait_out(n0 - 2, 0)
        extract(0)
        fire_out(n0, 0)

        @pl.when(i < NPAIRS1 - 1)
        def _():
            fire_in(n0 + 2, 0)
        drain_in(n0 + 1, 1)

        @pl.when(i > 0)
        def _():
            wait_out(n0 - 1, 1)
        extract(1)
        fire_out(n0 + 1, 1)
        return carry

    lax.fori_loop(0, NPAIRS1, pair_body, 0)
    wait_out(2 * NPAIRS1 - 2, 0)
    wait_out(2 * NPAIRS1 - 1, 1)

    @pl.when(w < 4)
    def _():
        fire_in(244, 0)
        drain_in(244, 0)
        extract(0)
        fire_out(244, 0)
        wait_out(244, 0)

    @pl.when(w == 4)
    def _():
        pltpu.sync_copy(tail_hbm, out0.at[pl.ds(0, 16), :])
        pltpu.sync_copy(out0.at[pl.ds(0, 16), :],
                        t4_hbm.at[pl.ds(NTILE_FULL * 32, 16), :])


def _gather_kernel(tl_hbm, idx_hbm, out_hbm,
                   xi_v, r0, r1, rows0, rows1, stage0, stage1,
                   gsem0, gsem1, ssem0, ssem1):
    w = lax.axis_index("s") * 2 + lax.axis_index("c")
    h = w // 16
    bg = w % 16
    f0 = h * NF
    rbuf = (r0, r1)
    rows = (rows0, rows1)
    gsem = (gsem0, gsem1)
    stage = (stage0, stage1)
    ssem = (ssem0, ssem1)
    iota = lax.iota(jnp.int32, 16)

    def build_r(f, pb):
        r_v = rbuf[pb]

        @plsc.parallel_loop(0, BBLK // 16, unroll=4)
        def _(g):
            g16 = pl.multiple_of(g * 16, 16)
            p16 = (iota + g16) * FIELDS + (f0 + f)
            r_v[pl.ds(g16, 16)] = plsc.load_gather(xi_v, [p16])

    def issue(pb):
        pltpu.async_copy(tl_hbm.at[rbuf[pb]], rows[pb], gsem[pb])

    def wait(pb):
        pltpu.make_async_copy(tl_hbm.at[rbuf[pb]], rows[pb], gsem[pb]).wait()

    def wait_stage(bt, pb):
        pltpu.make_async_copy(stage[pb], out_hbm.at[f0, :, bt, :, :],
                              ssem[pb]).wait()

    def extract(f, bt, pb):
        rows_b = rows[pb]
        stage_v = stage[pb]

        @plsc.parallel_loop(0, DIM, unroll=4)
        def _(d):
            d16 = jnp.full((16,), 0, jnp.int32) + d
            dr = lax.shift_right_logical(d, 3)
            dj = jnp.bitwise_and(d, 7)
            for g in range(BBLK // 16):
                stage_v[dr, dj, pl.ds(g * 16, 16)] = plsc.load_gather(
                    rows_b, [iota + g * 16, d16])
        pltpu.async_copy(stage_v, out_hbm.at[f0 + f, :, bt, :, :], ssem[pb])

    def block_body(blk, carry):
        bb0 = bg * (NBLK * BBLK) + blk * BBLK
        bt = bg * NBLK + blk
        pltpu.sync_copy(idx_hbm.at[pl.ds(bb0 * FIELDS, BBLK * FIELDS)], xi_v)

        build_r(0, 0)
        issue(0)

        def pair_body(i, carry2):
            c = 2 * i
            build_r(c + 1, 1)
            issue(1)
            wait(0)

            @pl.when(i > 0)
            def _():
                wait_stage(bt, 0)
            extract(c, bt, 0)
            build_r(c + 2, 0)
            issue(0)
            wait(1)

            @pl.when(i > 0)
            def _():
                wait_stage(bt, 1)
            extract(c + 1, bt, 1)
            return carry2
        lax.fori_loop(0, NPAIR2, pair_body, 0)

        wait(0)
        wait_stage(bt, 0)
        extract(NF - 1, bt, 0)
        wait_stage(bt, 0)
        wait_stage(bt, 1)
        return carry

    lax.fori_loop(0, NBLK, block_body, 0)


def kernel(x, embedding):
    batch, fields = x.shape
    b = batch * fields
    tt = jnp.transpose(embedding)
    mesh = plsc.VectorSubcoreMesh(core_axis_name="c", subcore_axis_name="s")

    k1 = pl.kernel(
        _transpose_kernel,
        out_type=jax.ShapeDtypeStruct((250000, 128), jnp.float32),
        mesh=mesh,
        scratch_types=[
            pltpu.VMEM((32, 128), jnp.float32),
            pltpu.VMEM((32, 128), jnp.float32),
            pltpu.VMEM((32, 128), jnp.float32),
            pltpu.VMEM((32, 128), jnp.float32),
            pltpu.SemaphoreType.DMA,
            pltpu.SemaphoreType.DMA,
            pltpu.SemaphoreType.DMA,
            pltpu.SemaphoreType.DMA,
        ],
        compiler_params=pltpu.CompilerParams(
            use_tc_tiling_on_sc=True, needs_layout_passes=False),
    )
    tail16 = embedding[NTILE_FULL * 128:, :].reshape(16, 128)
    t4 = k1(tt, tail16)
    tl = t4.reshape(NCAT, DIM)

    k2 = pl.kernel(
        _gather_kernel,
        out_type=jax.ShapeDtypeStruct((FIELDS, 4, 128, 8, 128), jnp.float32),
        mesh=mesh,
        scratch_types=[
            pltpu.VMEM((BBLK * FIELDS,), jnp.int32),
            pltpu.VMEM((BBLK,), jnp.int32),
            pltpu.VMEM((BBLK,), jnp.int32),
            pltpu.VMEM((BBLK, DIM), jnp.float32),
            pltpu.VMEM((BBLK, DIM), jnp.float32),
            pltpu.VMEM((4, 8, BBLK), jnp.float32),
            pltpu.VMEM((4, 8, BBLK), jnp.float32),
            pltpu.SemaphoreType.DMA,
            pltpu.SemaphoreType.DMA,
            pltpu.SemaphoreType.DMA,
            pltpu.SemaphoreType.DMA,
        ],
        compiler_params=pltpu.CompilerParams(
            use_tc_tiling_on_sc=False, needs_layout_passes=False),
    )
    out5 = k2(tl, x.reshape(b))
    return jnp.transpose(out5, (2, 4, 0, 1, 3)).reshape(batch, fields, DIM)
